# Initial kernel scaffold; baseline (speedup 1.0000x reference)
import jax
import jax.numpy as jnp
from jax.experimental import pallas as pl


def kernel(x, edge_index, emb, Wl0, bl0, Wr0, Wl1, bl1, Wr1, Wl2, bl2, Wr2, W_last, b_last):
    raise NotImplementedError("write your pallas kernel here")


# speedup vs baseline: 8.1186x; 8.1186x over previous
import functools

import jax
import jax.numpy as jnp
from jax import lax
from jax.experimental import pallas as pl
from jax.experimental.pallas import tpu as pltpu
from jax.experimental.pallas import tpu_sc as plsc

N = 10000
E = 320000
D = 128
V = 10000

NC = 2
NS = 16
NW = NC * NS

NP = 10240
EK = 80
ECHUNKS = E // (NW * EK)
PHASES = 5
PH = ECHUNKS // PHASES
XPT = NP // (NW * EK)
RPT = NP // NS

_mesh = plsc.VectorSubcoreMesh(
    core_axis_name="c", subcore_axis_name="s", num_cores=NC, num_subcores=NS
)


def _fill_const(ref, nrows, width, value):
    groups = width // 16
    vec = jnp.full((16,), value, jnp.float32)

    def body(i, _):
        r = i // groups
        g = i % groups
        ref[r, pl.ds(g * 16, 16)] = vec
        return 0

    lax.fori_loop(0, nrows * groups, body, 0)


@functools.partial(
    pl.kernel,
    out_type=(
        jax.ShapeDtypeStruct((NP, D), jnp.float32),
        jax.ShapeDtypeStruct((NC, NP, D), jnp.float32),
    ),
    mesh=_mesh,
    scratch_types=[
        pltpu.VMEM((XPT, EK), jnp.int32),
        pltpu.VMEM((EK, D), jnp.float32),
        pltpu.VMEM((PH, EK), jnp.int32),
        pltpu.VMEM_SHARED((NP, D), jnp.float32),
        pltpu.SemaphoreType.DMA,
    ],
)
def _sc_prep(x3d_hbm, emb_hbm, dst4d_hbm, h0_hbm, cnt_hbm,
             idx_v, rows_v, dst_v, cnt_sp, sem):
    c = lax.axis_index("c")
    s = lax.axis_index("s")
    w = s * NC + c

    _fill_const(rows_v, EK, D, 0.0)

    def zbody(t, _):
        pltpu.sync_copy(rows_v, cnt_sp.at[pl.ds(s * RPT + t * EK, EK)])
        return 0

    lax.fori_loop(0, RPT // EK, zbody, 0)

    pltpu.sync_copy(x3d_hbm.at[w], idx_v)
    for j in range(XPT):
        pltpu.async_copy(emb_hbm.at[idx_v.at[j]], rows_v, sem).wait()
        pltpu.sync_copy(rows_v, h0_hbm.at[pl.ds((w * XPT + j) * EK, EK)])

    _fill_const(rows_v, EK, D, 1.0)
    plsc.subcore_barrier()

    def phase(p, _):
        pltpu.sync_copy(dst4d_hbm.at[w, p], dst_v)

        def body(j, _):
            pltpu.sync_copy(rows_v, cnt_sp.at[dst_v.at[j]], add=True)
            return 0

        lax.fori_loop(0, PH, body, 0)
        return 0

    lax.fori_loop(0, PHASES, phase, 0)
    plsc.subcore_barrier()
    pltpu.sync_copy(
        cnt_sp.at[pl.ds(s * RPT, RPT)],
        cnt_hbm.at[c, pl.ds(s * RPT, RPT)],
    )


@functools.partial(
    pl.kernel,
    out_type=jax.ShapeDtypeStruct((NC, NP, D), jnp.float32),
    mesh=_mesh,
    scratch_types=[
        pltpu.VMEM((PH, EK), jnp.int32),
        pltpu.VMEM((PH, EK), jnp.int32),
        pltpu.VMEM((EK, D), jnp.float32),
        pltpu.VMEM((EK, D), jnp.float32),
        pltpu.VMEM((16, D), jnp.float32),
        pltpu.VMEM_SHARED((NP, D), jnp.float32),
        pltpu.SemaphoreType.DMA,
        pltpu.SemaphoreType.DMA,
    ],
)
def _sc_segsum(h_hbm, src4d_hbm, dst4d_hbm, agg_hbm,
               src_v, dst_v, rows0_v, rows1_v, zb_v, agg_sp, sem0, sem1):
    c = lax.axis_index("c")
    s = lax.axis_index("s")
    w = s * NC + c

    _fill_const(zb_v, 16, D, 0.0)

    def zbody(t, _):
        pltpu.sync_copy(zb_v, agg_sp.at[pl.ds(s * RPT + t * 16, 16)])
        return 0

    lax.fori_loop(0, RPT // 16, zbody, 0)
    plsc.subcore_barrier()

    bufs = (rows0_v, rows1_v)
    sems = (sem0, sem1)

    def phase(p, _):
        pltpu.sync_copy(src4d_hbm.at[w, p], src_v)
        pltpu.sync_copy(dst4d_hbm.at[w, p], dst_v)

        pltpu.async_copy(h_hbm.at[src_v.at[0]], rows0_v, sem0)

        def body(j, _):
            cur = j % 2
            for b in range(2):
                @pl.when(cur == b)
                def _():
                    @pl.when(j + 1 < PH)
                    def _():
                        pltpu.async_copy(
                            h_hbm.at[src_v.at[j + 1]], bufs[1 - b], sems[1 - b]
                        )
                    pltpu.make_async_copy(
                        h_hbm.at[src_v.at[j]], bufs[b], sems[b]
                    ).wait()
                    pltpu.sync_copy(bufs[b], agg_sp.at[dst_v.at[j]], add=True)
            return 0

        lax.fori_loop(0, PH, body, 0)
        return 0

    lax.fori_loop(0, PHASES, phase, 0)
    plsc.subcore_barrier()
    pltpu.sync_copy(
        agg_sp.at[pl.ds(s * RPT, RPT)],
        agg_hbm.at[c, pl.ds(s * RPT, RPT)],
    )


_RB = 2048


def _tc_layer_body(agg_ref, cnt_ref, h_ref, wl_ref, bl_ref, wr_ref, out_ref):
    agg = agg_ref[0] + agg_ref[1]
    deg = cnt_ref[0, :, 0:1] + cnt_ref[1, :, 0:1]
    inv = 1.0 / jnp.maximum(deg, 1.0)
    a = agg * inv
    ml = lax.dot_general(a, wl_ref[...], (((1,), (1,)), ((), ())),
                         preferred_element_type=jnp.float32)
    mr = lax.dot_general(h_ref[...], wr_ref[...], (((1,), (1,)), ((), ())),
                         preferred_element_type=jnp.float32)
    out_ref[...] = jnp.maximum(ml + mr + bl_ref[...], 0.0)


def _tc_layer(aggp, cnt, h, wl, bl2d, wr):
    return pl.pallas_call(
        _tc_layer_body,
        grid=(NP // _RB,),
        in_specs=[
            pl.BlockSpec((NC, _RB, D), lambda i: (0, i, 0)),
            pl.BlockSpec((NC, _RB, D), lambda i: (0, i, 0)),
            pl.BlockSpec((_RB, D), lambda i: (i, 0)),
            pl.BlockSpec((D, D), lambda i: (0, 0)),
            pl.BlockSpec((1, D), lambda i: (0, 0)),
            pl.BlockSpec((D, D), lambda i: (0, 0)),
        ],
        out_specs=pl.BlockSpec((_RB, D), lambda i: (i, 0)),
        out_shape=jax.ShapeDtypeStruct((NP, D), jnp.float32),
    )(aggp, cnt, h, wl, bl2d, wr)


_FRB = 2000
_FCB = 1024


def _tc_final_body(h_ref, w_ref, b_ref, out_ref):
    m = lax.dot_general(h_ref[...], w_ref[...], (((1,), (1,)), ((), ())),
                        preferred_element_type=jnp.float32)
    out_ref[...] = m + b_ref[...]


def _tc_final(h, w_last, b2d):
    ncb = (V + _FCB - 1) // _FCB
    return pl.pallas_call(
        _tc_final_body,
        grid=(N // _FRB, ncb),
        in_specs=[
            pl.BlockSpec((_FRB, D), lambda i, j: (i, 0)),
            pl.BlockSpec((_FCB, D), lambda i, j: (j, 0)),
            pl.BlockSpec((1, _FCB), lambda i, j: (0, j)),
        ],
        out_specs=pl.BlockSpec((_FRB, _FCB), lambda i, j: (i, j)),
        out_shape=jax.ShapeDtypeStruct((N, V), jnp.float32),
    )(h, w_last, b2d)


def kernel(x, edge_index, emb, Wl0, bl0, Wr0, Wl1, bl1, Wr1, Wl2, bl2, Wr2,
           W_last, b_last):
    xp = jnp.pad(x.astype(jnp.int32), (0, NP - N)).reshape(NW, XPT, EK)
    ei = edge_index.astype(jnp.int32)
    src4d = ei[0].reshape(NW, PHASES, PH, EK)
    dst4d = ei[1].reshape(NW, PHASES, PH, EK)

    h, cnt = _sc_prep(xp, emb, dst4d)
    for wl, bl, wr in ((Wl0, bl0, Wr0), (Wl1, bl1, Wr1), (Wl2, bl2, Wr2)):
        aggp = _sc_segsum(h, src4d, dst4d)
        h = _tc_layer(aggp, cnt, h, wl, bl.reshape(1, D), wr)
    return _tc_final(h, W_last, b_last.reshape(1, V))


# speedup vs baseline: 8.8090x; 1.0851x over previous
import functools

import jax
import jax.numpy as jnp
from jax import lax
from jax.experimental import pallas as pl
from jax.experimental.pallas import tpu as pltpu
from jax.experimental.pallas import tpu_sc as plsc

N = 10000
E = 320000
D = 128
V = 10000

NC = 2
NS = 16
NW = NC * NS

NP = 10240
EK = 80
ECHUNKS = E // (NW * EK)
PHASES = 5
PH = ECHUNKS // PHASES
XPT = NP // (NW * EK)
RPT = NP // NS

_mesh = plsc.VectorSubcoreMesh(
    core_axis_name="c", subcore_axis_name="s", num_cores=NC, num_subcores=NS
)


def _fill_const(ref, nrows, width, value):
    groups = width // 16
    vec = jnp.full((16,), value, jnp.float32)

    def body(i, _):
        r = i // groups
        g = i % groups
        ref[r, pl.ds(g * 16, 16)] = vec
        return 0

    lax.fori_loop(0, nrows * groups, body, 0)


@functools.partial(
    pl.kernel,
    out_type=(
        jax.ShapeDtypeStruct((NP, D), jnp.float32),
        jax.ShapeDtypeStruct((NC, NP, D), jnp.float32),
    ),
    mesh=_mesh,
    scratch_types=[
        pltpu.VMEM((XPT, EK), jnp.int32),
        pltpu.VMEM((EK, D), jnp.float32),
        pltpu.VMEM((PH, EK), jnp.int32),
        pltpu.VMEM_SHARED((NP, D), jnp.float32),
        pltpu.SemaphoreType.DMA,
        pltpu.SemaphoreType.DMA,
        pltpu.SemaphoreType.DMA,
    ],
)
def _sc_prep(x3d_hbm, emb_hbm, dst4d_hbm, h0_hbm, cnt_hbm,
             idx_v, rows_v, dst_v, cnt_sp, sem, csem0, csem1):
    c = lax.axis_index("c")
    s = lax.axis_index("s")
    w = s * NC + c

    _fill_const(rows_v, EK, D, 0.0)

    def zbody(t, _):
        pltpu.sync_copy(rows_v, cnt_sp.at[pl.ds(s * RPT + t * EK, EK)])
        return 0

    lax.fori_loop(0, RPT // EK, zbody, 0)

    pltpu.sync_copy(x3d_hbm.at[w], idx_v)
    for j in range(XPT):
        pltpu.async_copy(emb_hbm.at[idx_v.at[j]], rows_v, sem).wait()
        pltpu.sync_copy(rows_v, h0_hbm.at[pl.ds((w * XPT + j) * EK, EK)])

    _fill_const(rows_v, EK, D, 1.0)
    plsc.subcore_barrier()
    csems = (csem0, csem1)

    def phase(p, _):
        pltpu.sync_copy(dst4d_hbm.at[w, p], dst_v)

        def body(j, _):
            for b in range(2):
                @pl.when(j % 2 == b)
                def _():
                    @pl.when(j >= 2)
                    def _():
                        pltpu.make_async_copy(
                            rows_v, cnt_sp.at[dst_v.at[j - 2]], csems[b]
                        ).wait()
                    pltpu.async_copy(
                        rows_v, cnt_sp.at[dst_v.at[j]], csems[b], add=True
                    )
            return 0

        lax.fori_loop(0, PH, body, 0)
        pltpu.make_async_copy(
            rows_v, cnt_sp.at[dst_v.at[PH - 2]], csems[(PH - 2) % 2]
        ).wait()
        pltpu.make_async_copy(
            rows_v, cnt_sp.at[dst_v.at[PH - 1]], csems[(PH - 1) % 2]
        ).wait()
        return 0

    lax.fori_loop(0, PHASES, phase, 0)
    plsc.subcore_barrier()
    pltpu.sync_copy(
        cnt_sp.at[pl.ds(s * RPT, RPT)],
        cnt_hbm.at[c, pl.ds(s * RPT, RPT)],
    )


@functools.partial(
    pl.kernel,
    out_type=jax.ShapeDtypeStruct((NC, NP, D), jnp.float32),
    mesh=_mesh,
    scratch_types=[
        pltpu.VMEM((PH, EK), jnp.int32),
        pltpu.VMEM((PH, EK), jnp.int32),
        pltpu.VMEM((EK, D), jnp.float32),
        pltpu.VMEM((EK, D), jnp.float32),
        pltpu.VMEM((EK, D), jnp.float32),
        pltpu.VMEM((8, D), jnp.float32),
        pltpu.VMEM_SHARED((NP, D), jnp.float32),
        pltpu.SemaphoreType.DMA,
        pltpu.SemaphoreType.DMA,
        pltpu.SemaphoreType.DMA,
        pltpu.SemaphoreType.DMA,
        pltpu.SemaphoreType.DMA,
        pltpu.SemaphoreType.DMA,
    ],
)
def _sc_segsum(h_hbm, src4d_hbm, dst4d_hbm, agg_hbm,
               src_v, dst_v, rows0_v, rows1_v, rows2_v, zb_v, agg_sp,
               gsem0, gsem1, gsem2, ssem0, ssem1, ssem2):
    c = lax.axis_index("c")
    s = lax.axis_index("s")
    w = s * NC + c

    _fill_const(zb_v, 8, D, 0.0)

    def zbody(t, _):
        pltpu.sync_copy(zb_v, agg_sp.at[pl.ds(s * RPT + t * 8, 8)])
        return 0

    lax.fori_loop(0, RPT // 8, zbody, 0)
    plsc.subcore_barrier()

    bufs = (rows0_v, rows1_v, rows2_v)
    gsems = (gsem0, gsem1, gsem2)
    ssems = (ssem0, ssem1, ssem2)

    def phase(p, _):
        pltpu.sync_copy(src4d_hbm.at[w, p], src_v)
        pltpu.sync_copy(dst4d_hbm.at[w, p], dst_v)

        pltpu.async_copy(h_hbm.at[src_v.at[0]], bufs[0], gsems[0])
        pltpu.async_copy(h_hbm.at[src_v.at[1]], bufs[1], gsems[1])

        def body(j, _):
            for b in range(3):
                @pl.when(j % 3 == b)
                def _():
                    nb = (b + 2) % 3

                    @pl.when(j >= 1)
                    def _():
                        pltpu.make_async_copy(
                            bufs[nb], agg_sp.at[dst_v.at[j - 1]], ssems[nb]
                        ).wait()

                    @pl.when(j + 2 < PH)
                    def _():
                        pltpu.async_copy(
                            h_hbm.at[src_v.at[j + 2]], bufs[nb], gsems[nb]
                        )
                    pltpu.make_async_copy(
                        h_hbm.at[src_v.at[j]], bufs[b], gsems[b]
                    ).wait()
                    pltpu.async_copy(
                        bufs[b], agg_sp.at[dst_v.at[j]], ssems[b], add=True
                    )
            return 0

        lax.fori_loop(0, PH, body, 0)
        pltpu.make_async_copy(
            bufs[(PH - 1) % 3], agg_sp.at[dst_v.at[PH - 1]], ssems[(PH - 1) % 3]
        ).wait()
        return 0

    lax.fori_loop(0, PHASES, phase, 0)
    plsc.subcore_barrier()
    pltpu.sync_copy(
        agg_sp.at[pl.ds(s * RPT, RPT)],
        agg_hbm.at[c, pl.ds(s * RPT, RPT)],
    )


_RB = 2048


def _tc_layer_body(agg_ref, cnt_ref, h_ref, wl_ref, bl_ref, wr_ref, out_ref):
    agg = agg_ref[0] + agg_ref[1]
    deg = cnt_ref[0, :, 0:1] + cnt_ref[1, :, 0:1]
    inv = 1.0 / jnp.maximum(deg, 1.0)
    a = agg * inv
    ml = lax.dot_general(a, wl_ref[...], (((1,), (1,)), ((), ())),
                         preferred_element_type=jnp.float32)
    mr = lax.dot_general(h_ref[...], wr_ref[...], (((1,), (1,)), ((), ())),
                         preferred_element_type=jnp.float32)
    out_ref[...] = jnp.maximum(ml + mr + bl_ref[...], 0.0)


def _tc_layer(aggp, cnt, h, wl, bl2d, wr):
    return pl.pallas_call(
        _tc_layer_body,
        grid=(NP // _RB,),
        in_specs=[
            pl.BlockSpec((NC, _RB, D), lambda i: (0, i, 0)),
            pl.BlockSpec((NC, _RB, D), lambda i: (0, i, 0)),
            pl.BlockSpec((_RB, D), lambda i: (i, 0)),
            pl.BlockSpec((D, D), lambda i: (0, 0)),
            pl.BlockSpec((1, D), lambda i: (0, 0)),
            pl.BlockSpec((D, D), lambda i: (0, 0)),
        ],
        out_specs=pl.BlockSpec((_RB, D), lambda i: (i, 0)),
        out_shape=jax.ShapeDtypeStruct((NP, D), jnp.float32),
    )(aggp, cnt, h, wl, bl2d, wr)


_FRB = 2000
_FCB = 1024


def _tc_final_body(h_ref, w_ref, b_ref, out_ref):
    m = lax.dot_general(h_ref[...], w_ref[...], (((1,), (1,)), ((), ())),
                        preferred_element_type=jnp.float32)
    out_ref[...] = m + b_ref[...]


def _tc_final(h, w_last, b2d):
    ncb = (V + _FCB - 1) // _FCB
    return pl.pallas_call(
        _tc_final_body,
        grid=(N // _FRB, ncb),
        in_specs=[
            pl.BlockSpec((_FRB, D), lambda i, j: (i, 0)),
            pl.BlockSpec((_FCB, D), lambda i, j: (j, 0)),
            pl.BlockSpec((1, _FCB), lambda i, j: (0, j)),
        ],
        out_specs=pl.BlockSpec((_FRB, _FCB), lambda i, j: (i, j)),
        out_shape=jax.ShapeDtypeStruct((N, V), jnp.float32),
    )(h, w_last, b2d)


def kernel(x, edge_index, emb, Wl0, bl0, Wr0, Wl1, bl1, Wr1, Wl2, bl2, Wr2,
           W_last, b_last):
    xp = jnp.pad(x.astype(jnp.int32), (0, NP - N)).reshape(NW, XPT, EK)
    ei = edge_index.astype(jnp.int32)
    src4d = ei[0].reshape(NW, PHASES, PH, EK)
    dst4d = ei[1].reshape(NW, PHASES, PH, EK)

    h, cnt = _sc_prep(xp, emb, dst4d)
    for wl, bl, wr in ((Wl0, bl0, Wr0), (Wl1, bl1, Wr1), (Wl2, bl2, Wr2)):
        aggp = _sc_segsum(h, src4d, dst4d)
        h = _tc_layer(aggp, cnt, h, wl, bl.reshape(1, D), wr)
    return _tc_final(h, W_last, b_last.reshape(1, V))


# speedup vs baseline: 8.8348x; 1.0029x over previous
import functools

import jax
import jax.numpy as jnp
from jax import lax
from jax.experimental import pallas as pl
from jax.experimental.pallas import tpu as pltpu
from jax.experimental.pallas import tpu_sc as plsc

N = 10000
E = 320000
D = 128
V = 10000

NC = 2
NS = 16
NW = NC * NS

NP = 10240
EK = 80
ECHUNKS = E // (NW * EK)
PHASES = 5
PH = ECHUNKS // PHASES
XPT = NP // (NW * EK)
RPT = NP // NS

_mesh = plsc.VectorSubcoreMesh(
    core_axis_name="c", subcore_axis_name="s", num_cores=NC, num_subcores=NS
)


def _fill_const(ref, nrows, width, value):
    groups = width // 16
    vec = jnp.full((16,), value, jnp.float32)

    def body(i, _):
        r = i // groups
        g = i % groups
        ref[r, pl.ds(g * 16, 16)] = vec
        return 0

    lax.fori_loop(0, nrows * groups, body, 0)


@functools.partial(
    pl.kernel,
    out_type=(
        jax.ShapeDtypeStruct((NP, D), jnp.float32),
        jax.ShapeDtypeStruct((NC, NP, D), jnp.float32),
    ),
    mesh=_mesh,
    scratch_types=[
        pltpu.VMEM((XPT, EK), jnp.int32),
        pltpu.VMEM((EK, D), jnp.float32),
        pltpu.VMEM((PH, EK), jnp.int32),
        pltpu.VMEM_SHARED((NP, D), jnp.float32),
        pltpu.SemaphoreType.DMA,
        pltpu.SemaphoreType.DMA,
        pltpu.SemaphoreType.DMA,
    ],
)
def _sc_prep(x3d_hbm, emb_hbm, dst4d_hbm, h0_hbm, cnt_hbm,
             idx_v, rows_v, dst_v, cnt_sp, sem, csem0, csem1):
    c = lax.axis_index("c")
    s = lax.axis_index("s")
    w = s * NC + c

    _fill_const(rows_v, EK, D, 0.0)

    def zbody(t, _):
        pltpu.sync_copy(rows_v, cnt_sp.at[pl.ds(s * RPT + t * EK, EK)])
        return 0

    lax.fori_loop(0, RPT // EK, zbody, 0)

    pltpu.sync_copy(x3d_hbm.at[w], idx_v)
    for j in range(XPT):
        pltpu.async_copy(emb_hbm.at[idx_v.at[j]], rows_v, sem).wait()
        pltpu.sync_copy(rows_v, h0_hbm.at[pl.ds((w * XPT + j) * EK, EK)])

    _fill_const(rows_v, EK, D, 1.0)
    plsc.subcore_barrier()
    csems = (csem0, csem1)

    def phase(p, _):
        pltpu.sync_copy(dst4d_hbm.at[w, p], dst_v)

        def body(j, _):
            for b in range(2):
                @pl.when(j % 2 == b)
                def _():
                    @pl.when(j >= 2)
                    def _():
                        pltpu.make_async_copy(
                            rows_v, cnt_sp.at[dst_v.at[j - 2]], csems[b]
                        ).wait()
                    pltpu.async_copy(
                        rows_v, cnt_sp.at[dst_v.at[j]], csems[b], add=True
                    )
            return 0

        lax.fori_loop(0, PH, body, 0)
        pltpu.make_async_copy(
            rows_v, cnt_sp.at[dst_v.at[PH - 2]], csems[(PH - 2) % 2]
        ).wait()
        pltpu.make_async_copy(
            rows_v, cnt_sp.at[dst_v.at[PH - 1]], csems[(PH - 1) % 2]
        ).wait()
        return 0

    lax.fori_loop(0, PHASES, phase, 0)
    plsc.subcore_barrier()
    pltpu.sync_copy(
        cnt_sp.at[pl.ds(s * RPT, RPT)],
        cnt_hbm.at[c, pl.ds(s * RPT, RPT)],
    )


@functools.partial(
    pl.kernel,
    out_type=jax.ShapeDtypeStruct((NC, NP, D), jnp.float32),
    mesh=_mesh,
    scratch_types=[
        pltpu.VMEM((PH, EK), jnp.int32),
        pltpu.VMEM((PH, EK), jnp.int32),
        pltpu.VMEM((EK, D), jnp.float32),
        pltpu.VMEM((EK, D), jnp.float32),
        pltpu.VMEM((EK, D), jnp.float32),
        pltpu.VMEM((8, D), jnp.float32),
        pltpu.VMEM_SHARED((NP, D), jnp.float32),
        pltpu.SemaphoreType.DMA,
        pltpu.SemaphoreType.DMA,
        pltpu.SemaphoreType.DMA,
        pltpu.SemaphoreType.DMA,
        pltpu.SemaphoreType.DMA,
        pltpu.SemaphoreType.DMA,
    ],
)
def _sc_segsum(h_hbm, src4d_hbm, dst4d_hbm, agg_hbm,
               src_v, dst_v, rows0_v, rows1_v, rows2_v, zb_v, agg_sp,
               gsem0, gsem1, gsem2, ssem0, ssem1, ssem2):
    c = lax.axis_index("c")
    s = lax.axis_index("s")
    w = s * NC + c

    _fill_const(zb_v, 8, D, 0.0)

    def zbody(t, _):
        pltpu.sync_copy(zb_v, agg_sp.at[pl.ds(s * RPT + t * 8, 8)])
        return 0

    lax.fori_loop(0, RPT // 8, zbody, 0)
    plsc.subcore_barrier()

    bufs = (rows0_v, rows1_v, rows2_v)
    gsems = (gsem0, gsem1, gsem2)
    ssems = (ssem0, ssem1, ssem2)

    def phase(p, _):
        pltpu.sync_copy(src4d_hbm.at[w, p], src_v)
        pltpu.sync_copy(dst4d_hbm.at[w, p], dst_v)

        pltpu.async_copy(h_hbm.at[src_v.at[0]], bufs[0], gsems[0])
        pltpu.async_copy(h_hbm.at[src_v.at[1]], bufs[1], gsems[1])

        def body(j, _):
            for b in range(3):
                @pl.when(j % 3 == b)
                def _():
                    nb = (b + 2) % 3

                    @pl.when(j >= 1)
                    def _():
                        pltpu.make_async_copy(
                            bufs[nb], agg_sp.at[dst_v.at[j - 1]], ssems[nb]
                        ).wait()

                    @pl.when(j + 2 < PH)
                    def _():
                        pltpu.async_copy(
                            h_hbm.at[src_v.at[j + 2]], bufs[nb], gsems[nb]
                        )
                    pltpu.make_async_copy(
                        h_hbm.at[src_v.at[j]], bufs[b], gsems[b]
                    ).wait()
                    pltpu.async_copy(
                        bufs[b], agg_sp.at[dst_v.at[j]], ssems[b], add=True
                    )
            return 0

        lax.fori_loop(0, PH, body, 0)
        pltpu.make_async_copy(
            bufs[(PH - 1) % 3], agg_sp.at[dst_v.at[PH - 1]], ssems[(PH - 1) % 3]
        ).wait()
        return 0

    lax.fori_loop(0, PHASES, phase, 0)
    plsc.subcore_barrier()
    pltpu.sync_copy(
        agg_sp.at[pl.ds(s * RPT, RPT)],
        agg_hbm.at[c, pl.ds(s * RPT, RPT)],
    )


_RB = 2048


def _tc_invd_body(cnt_ref, out_ref):
    deg = cnt_ref[0, :, 0:1] + cnt_ref[1, :, 0:1]
    out_ref[...] = 1.0 / jnp.maximum(deg, 1.0)


def _tc_invd(cnt):
    return pl.pallas_call(
        _tc_invd_body,
        grid=(NP // _RB,),
        in_specs=[pl.BlockSpec((NC, _RB, D), lambda i: (0, i, 0))],
        out_specs=pl.BlockSpec((_RB, 1), lambda i: (i, 0)),
        out_shape=jax.ShapeDtypeStruct((NP, 1), jnp.float32),
    )(cnt)


def _tc_r_body(h_ref, wr_ref, bl_ref, out_ref):
    mr = lax.dot_general(h_ref[...], wr_ref[...], (((1,), (1,)), ((), ())),
                         preferred_element_type=jnp.float32)
    out_ref[...] = mr + bl_ref[...]


def _tc_r(h, wr, bl2d):
    return pl.pallas_call(
        _tc_r_body,
        grid=(NP // _RB,),
        in_specs=[
            pl.BlockSpec((_RB, D), lambda i: (i, 0)),
            pl.BlockSpec((D, D), lambda i: (0, 0)),
            pl.BlockSpec((1, D), lambda i: (0, 0)),
        ],
        out_specs=pl.BlockSpec((_RB, D), lambda i: (i, 0)),
        out_shape=jax.ShapeDtypeStruct((NP, D), jnp.float32),
    )(h, wr, bl2d)


def _tc_combine_body(agg_ref, inv_ref, r_ref, wl_ref, out_ref):
    a = (agg_ref[0] + agg_ref[1]) * inv_ref[...]
    ml = lax.dot_general(a, wl_ref[...], (((1,), (1,)), ((), ())),
                         preferred_element_type=jnp.float32)
    out_ref[...] = jnp.maximum(ml + r_ref[...], 0.0)


def _tc_combine(aggp, inv, r, wl):
    return pl.pallas_call(
        _tc_combine_body,
        grid=(NP // _RB,),
        in_specs=[
            pl.BlockSpec((NC, _RB, D), lambda i: (0, i, 0)),
            pl.BlockSpec((_RB, 1), lambda i: (i, 0)),
            pl.BlockSpec((_RB, D), lambda i: (i, 0)),
            pl.BlockSpec((D, D), lambda i: (0, 0)),
        ],
        out_specs=pl.BlockSpec((_RB, D), lambda i: (i, 0)),
        out_shape=jax.ShapeDtypeStruct((NP, D), jnp.float32),
    )(aggp, inv, r, wl)


_FRB = 2000
_FCB = 1024


def _tc_final_body(h_ref, w_ref, b_ref, out_ref):
    m = lax.dot_general(h_ref[...], w_ref[...], (((1,), (1,)), ((), ())),
                        preferred_element_type=jnp.float32)
    out_ref[...] = m + b_ref[...]


def _tc_final(h, w_last, b2d):
    ncb = (V + _FCB - 1) // _FCB
    return pl.pallas_call(
        _tc_final_body,
        grid=(N // _FRB, ncb),
        in_specs=[
            pl.BlockSpec((_FRB, D), lambda i, j: (i, 0)),
            pl.BlockSpec((_FCB, D), lambda i, j: (j, 0)),
            pl.BlockSpec((1, _FCB), lambda i, j: (0, j)),
        ],
        out_specs=pl.BlockSpec((_FRB, _FCB), lambda i, j: (i, j)),
        out_shape=jax.ShapeDtypeStruct((N, V), jnp.float32),
    )(h, w_last, b2d)


def kernel(x, edge_index, emb, Wl0, bl0, Wr0, Wl1, bl1, Wr1, Wl2, bl2, Wr2,
           W_last, b_last):
    xp = jnp.pad(x.astype(jnp.int32), (0, NP - N)).reshape(NW, XPT, EK)
    ei = edge_index.astype(jnp.int32)
    src4d = ei[0].reshape(NW, PHASES, PH, EK)
    dst4d = ei[1].reshape(NW, PHASES, PH, EK)

    h, cnt = _sc_prep(xp, emb, dst4d)
    inv = _tc_invd(cnt)
    for wl, bl, wr in ((Wl0, bl0, Wr0), (Wl1, bl1, Wr1), (Wl2, bl2, Wr2)):
        aggp = _sc_segsum(h, src4d, dst4d)
        r = _tc_r(h, wr, bl.reshape(1, D))
        h = _tc_combine(aggp, inv, r, wl)
    return _tc_final(h, W_last, b_last.reshape(1, V))


# speedup vs baseline: 8.8664x; 1.0036x over previous
import functools

import jax
import jax.numpy as jnp
from jax import lax
from jax.experimental import pallas as pl
from jax.experimental.pallas import tpu as pltpu
from jax.experimental.pallas import tpu_sc as plsc

N = 10000
E = 320000
D = 128
V = 10000

NC = 2
NS = 16
NW = NC * NS

NP = 10240
EK = 80
ECHUNKS = E // (NW * EK)
PHASES = 5
PH = ECHUNKS // PHASES
XPT = NP // (NW * EK)
RPT = NP // NS

_mesh = plsc.VectorSubcoreMesh(
    core_axis_name="c", subcore_axis_name="s", num_cores=NC, num_subcores=NS
)


def _fill_const(ref, nrows, width, value):
    groups = width // 16
    vec = jnp.full((16,), value, jnp.float32)

    def body(i, _):
        r = i // groups
        g = i % groups
        ref[r, pl.ds(g * 16, 16)] = vec
        return 0

    lax.fori_loop(0, nrows * groups, body, 0)


@functools.partial(
    pl.kernel,
    out_type=(
        jax.ShapeDtypeStruct((NP, D), jnp.float32),
        jax.ShapeDtypeStruct((NC, NP, D), jnp.float32),
    ),
    mesh=_mesh,
    scratch_types=[
        pltpu.VMEM((XPT, EK), jnp.int32),
        pltpu.VMEM((EK, D), jnp.float32),
        pltpu.VMEM((EK, D), jnp.float32),
        pltpu.VMEM((EK, D), jnp.float32),
        pltpu.VMEM((PH, EK), jnp.int32),
        pltpu.VMEM_SHARED((NP, D), jnp.float32),
        pltpu.SemaphoreType.DMA,
        pltpu.SemaphoreType.DMA,
        pltpu.SemaphoreType.DMA,
        pltpu.SemaphoreType.DMA,
        pltpu.SemaphoreType.DMA,
    ],
)
def _sc_prep(x3d_hbm, emb_hbm, dst4d_hbm, h0_hbm, cnt_hbm,
             idx_v, rows0_v, rows1_v, ones_v, dst_v, cnt_sp,
             gsem0, gsem1, wsem0, csem0, csem1):
    c = lax.axis_index("c")
    s = lax.axis_index("s")
    w = s * NC + c

    _fill_const(ones_v, EK, D, 0.0)

    def zbody(t, _):
        pltpu.sync_copy(ones_v, cnt_sp.at[pl.ds(s * RPT + t * EK, EK)])
        return 0

    lax.fori_loop(0, RPT // EK, zbody, 0)

    pltpu.sync_copy(x3d_hbm.at[w], idx_v)
    gbufs = (rows0_v, rows1_v)
    gsems = (gsem0, gsem1)
    pltpu.async_copy(emb_hbm.at[idx_v.at[0]], rows0_v, gsem0)
    for j in range(XPT):
        b = j % 2
        if j + 1 < XPT:
            if j >= 1:
                pltpu.make_async_copy(
                    gbufs[1 - b],
                    h0_hbm.at[pl.ds((w * XPT + j - 1) * EK, EK)],
                    wsem0,
                ).wait()
            pltpu.async_copy(emb_hbm.at[idx_v.at[j + 1]], gbufs[1 - b],
                             gsems[1 - b])
        pltpu.make_async_copy(emb_hbm.at[idx_v.at[j]], gbufs[b],
                              gsems[b]).wait()
        pltpu.async_copy(gbufs[b], h0_hbm.at[pl.ds((w * XPT + j) * EK, EK)],
                         wsem0)
    pltpu.make_async_copy(
        gbufs[(XPT - 2) % 2],
        h0_hbm.at[pl.ds((w * XPT + XPT - 2) * EK, EK)], wsem0,
    ).wait()
    pltpu.make_async_copy(
        gbufs[(XPT - 1) % 2],
        h0_hbm.at[pl.ds((w * XPT + XPT - 1) * EK, EK)], wsem0,
    ).wait()

    _fill_const(ones_v, EK, D, 1.0)
    plsc.subcore_barrier()
    csems = (csem0, csem1)

    def phase(p, _):
        pltpu.sync_copy(dst4d_hbm.at[w, p], dst_v)

        def body(j, _):
            for b in range(2):
                @pl.when(j % 2 == b)
                def _():
                    @pl.when(j >= 2)
                    def _():
                        pltpu.make_async_copy(
                            ones_v, cnt_sp.at[dst_v.at[j - 2]], csems[b]
                        ).wait()
                    pltpu.async_copy(
                        ones_v, cnt_sp.at[dst_v.at[j]], csems[b], add=True
                    )
            return 0

        lax.fori_loop(0, PH, body, 0)
        pltpu.make_async_copy(
            ones_v, cnt_sp.at[dst_v.at[PH - 2]], csems[(PH - 2) % 2]
        ).wait()
        pltpu.make_async_copy(
            ones_v, cnt_sp.at[dst_v.at[PH - 1]], csems[(PH - 1) % 2]
        ).wait()
        return 0

    lax.fori_loop(0, PHASES, phase, 0)
    plsc.subcore_barrier()
    pltpu.sync_copy(
        cnt_sp.at[pl.ds(s * RPT, RPT)],
        cnt_hbm.at[c, pl.ds(s * RPT, RPT)],
    )


@functools.partial(
    pl.kernel,
    out_type=jax.ShapeDtypeStruct((NC, NP, D), jnp.float32),
    mesh=_mesh,
    scratch_types=[
        pltpu.VMEM((PH, EK), jnp.int32),
        pltpu.VMEM((PH, EK), jnp.int32),
        pltpu.VMEM((EK, D), jnp.float32),
        pltpu.VMEM((EK, D), jnp.float32),
        pltpu.VMEM((EK, D), jnp.float32),
        pltpu.VMEM((8, D), jnp.float32),
        pltpu.VMEM_SHARED((NP, D), jnp.float32),
        pltpu.SemaphoreType.DMA,
        pltpu.SemaphoreType.DMA,
        pltpu.SemaphoreType.DMA,
        pltpu.SemaphoreType.DMA,
        pltpu.SemaphoreType.DMA,
        pltpu.SemaphoreType.DMA,
    ],
)
def _sc_segsum(h_hbm, src4d_hbm, dst4d_hbm, agg_hbm,
               src_v, dst_v, rows0_v, rows1_v, rows2_v, zb_v, agg_sp,
               gsem0, gsem1, gsem2, ssem0, ssem1, ssem2):
    c = lax.axis_index("c")
    s = lax.axis_index("s")
    w = s * NC + c

    _fill_const(zb_v, 8, D, 0.0)

    def zbody(t, _):
        pltpu.sync_copy(zb_v, agg_sp.at[pl.ds(s * RPT + t * 8, 8)])
        return 0

    lax.fori_loop(0, RPT // 8, zbody, 0)
    plsc.subcore_barrier()

    bufs = (rows0_v, rows1_v, rows2_v)
    gsems = (gsem0, gsem1, gsem2)
    ssems = (ssem0, ssem1, ssem2)

    def phase(p, _):
        pltpu.sync_copy(src4d_hbm.at[w, p], src_v)
        pltpu.sync_copy(dst4d_hbm.at[w, p], dst_v)

        pltpu.async_copy(h_hbm.at[src_v.at[0]], bufs[0], gsems[0])
        pltpu.async_copy(h_hbm.at[src_v.at[1]], bufs[1], gsems[1])

        def body(j, _):
            for b in range(3):
                @pl.when(j % 3 == b)
                def _():
                    nb = (b + 2) % 3

                    @pl.when(j >= 1)
                    def _():
                        pltpu.make_async_copy(
                            bufs[nb], agg_sp.at[dst_v.at[j - 1]], ssems[nb]
                        ).wait()

                    @pl.when(j + 2 < PH)
                    def _():
                        pltpu.async_copy(
                            h_hbm.at[src_v.at[j + 2]], bufs[nb], gsems[nb]
                        )
                    pltpu.make_async_copy(
                        h_hbm.at[src_v.at[j]], bufs[b], gsems[b]
                    ).wait()
                    pltpu.async_copy(
                        bufs[b], agg_sp.at[dst_v.at[j]], ssems[b], add=True
                    )
            return 0

        lax.fori_loop(0, PH, body, 0)
        pltpu.make_async_copy(
            bufs[(PH - 1) % 3], agg_sp.at[dst_v.at[PH - 1]], ssems[(PH - 1) % 3]
        ).wait()
        return 0

    lax.fori_loop(0, PHASES, phase, 0)
    plsc.subcore_barrier()
    pltpu.sync_copy(
        agg_sp.at[pl.ds(s * RPT, RPT)],
        agg_hbm.at[c, pl.ds(s * RPT, RPT)],
    )


_RB = 2048


def _tc_invd_body(cnt_ref, out_ref):
    deg = cnt_ref[0, :, 0:1] + cnt_ref[1, :, 0:1]
    out_ref[...] = 1.0 / jnp.maximum(deg, 1.0)


def _tc_invd(cnt):
    return pl.pallas_call(
        _tc_invd_body,
        grid=(NP // _RB,),
        in_specs=[pl.BlockSpec((NC, _RB, D), lambda i: (0, i, 0))],
        out_specs=pl.BlockSpec((_RB, 1), lambda i: (i, 0)),
        out_shape=jax.ShapeDtypeStruct((NP, 1), jnp.float32),
    )(cnt)


def _tc_r_body(h_ref, wr_ref, bl_ref, out_ref):
    mr = lax.dot_general(h_ref[...], wr_ref[...], (((1,), (1,)), ((), ())),
                         preferred_element_type=jnp.float32)
    out_ref[...] = mr + bl_ref[...]


def _tc_r(h, wr, bl2d):
    return pl.pallas_call(
        _tc_r_body,
        grid=(NP // _RB,),
        in_specs=[
            pl.BlockSpec((_RB, D), lambda i: (i, 0)),
            pl.BlockSpec((D, D), lambda i: (0, 0)),
            pl.BlockSpec((1, D), lambda i: (0, 0)),
        ],
        out_specs=pl.BlockSpec((_RB, D), lambda i: (i, 0)),
        out_shape=jax.ShapeDtypeStruct((NP, D), jnp.float32),
    )(h, wr, bl2d)


def _tc_combine_body(agg_ref, inv_ref, r_ref, wl_ref, out_ref):
    a = (agg_ref[0] + agg_ref[1]) * inv_ref[...]
    ml = lax.dot_general(a, wl_ref[...], (((1,), (1,)), ((), ())),
                         preferred_element_type=jnp.float32)
    out_ref[...] = jnp.maximum(ml + r_ref[...], 0.0)


def _tc_combine(aggp, inv, r, wl):
    return pl.pallas_call(
        _tc_combine_body,
        grid=(NP // _RB,),
        in_specs=[
            pl.BlockSpec((NC, _RB, D), lambda i: (0, i, 0)),
            pl.BlockSpec((_RB, 1), lambda i: (i, 0)),
            pl.BlockSpec((_RB, D), lambda i: (i, 0)),
            pl.BlockSpec((D, D), lambda i: (0, 0)),
        ],
        out_specs=pl.BlockSpec((_RB, D), lambda i: (i, 0)),
        out_shape=jax.ShapeDtypeStruct((NP, D), jnp.float32),
    )(aggp, inv, r, wl)


_FRB = 2000
_FCB = 1024


def _tc_final_body(h_ref, w_ref, b_ref, out_ref):
    m = lax.dot_general(h_ref[...], w_ref[...], (((1,), (1,)), ((), ())),
                        preferred_element_type=jnp.float32)
    out_ref[...] = m + b_ref[...]


def _tc_final(h, w_last, b2d):
    ncb = (V + _FCB - 1) // _FCB
    return pl.pallas_call(
        _tc_final_body,
        grid=(N // _FRB, ncb),
        in_specs=[
            pl.BlockSpec((_FRB, D), lambda i, j: (i, 0)),
            pl.BlockSpec((_FCB, D), lambda i, j: (j, 0)),
            pl.BlockSpec((1, _FCB), lambda i, j: (0, j)),
        ],
        out_specs=pl.BlockSpec((_FRB, _FCB), lambda i, j: (i, j)),
        out_shape=jax.ShapeDtypeStruct((N, V), jnp.float32),
    )(h, w_last, b2d)


def kernel(x, edge_index, emb, Wl0, bl0, Wr0, Wl1, bl1, Wr1, Wl2, bl2, Wr2,
           W_last, b_last):
    xp = jnp.pad(x.astype(jnp.int32), (0, NP - N)).reshape(NW, XPT, EK)
    ei = edge_index.astype(jnp.int32)
    src4d = ei[0].reshape(NW, PHASES, PH, EK)
    dst4d = ei[1].reshape(NW, PHASES, PH, EK)

    h, cnt = _sc_prep(xp, emb, dst4d)
    inv = _tc_invd(cnt)
    for wl, bl, wr in ((Wl0, bl0, Wr0), (Wl1, bl1, Wr1), (Wl2, bl2, Wr2)):
        aggp = _sc_segsum(h, src4d, dst4d)
        r = _tc_r(h, wr, bl.reshape(1, D))
        h = _tc_combine(aggp, inv, r, wl)
    return _tc_final(h, W_last, b_last.reshape(1, V))
